# baseline (speedup 1.0000x reference)
import functools

import jax
import jax.numpy as jnp
from jax import lax
from jax.experimental import pallas as pl
from jax.experimental.pallas import tpu as pltpu
from jax.experimental.pallas import tpu_sc as plsc

NBINS = 256
START = -15.0
STEP = 30.0 / 256.0
INV_STEP = 256.0 / 30.0
B, N, C = 1024, 100, 4
LANES = 16


def _kde_body(rows_per_w, v_hbm, w_hbm, o0, o1, o2, o3,
              vv, wv, a0, a1, a2, a3, sem1, sem2):
    nc = 2
    wid = lax.axis_index("s") * nc + lax.axis_index("c")
    r0 = wid * rows_per_w

    v_words = N * rows_per_w
    w_words = C * N * rows_per_w
    outs = (o0, o1, o2, o3)
    accs = (a0, a1, a2, a3)

    cp1 = pltpu.async_copy(v_hbm.at[pl.ds(wid * v_words, v_words)], vv, sem1)
    cp2 = pltpu.async_copy(w_hbm.at[pl.ds(wid * w_words, w_words)], wv, sem2)

    zeros = jnp.zeros((LANES,), jnp.float32)

    def zbody(r, _):
        for acc in accs:
            for u in range(NBINS // LANES):
                acc[r, pl.ds(u * LANES, LANES)] = zeros
        return 0

    lax.fori_loop(0, rows_per_w, zbody, 0)

    cp1.wait()
    cp2.wait()

    iota = lax.iota(jnp.int32, LANES)
    for g in range(rows_per_w // LANES):
        rows = g * LANES + iota

        def step(n, g=g, rows=rows):
            v = vv[pl.ds(n * rows_per_w + g * LANES, LANES)]
            t = (v - START) * INV_STEP
            j = t.astype(jnp.int32)
            e_lo = j.astype(jnp.float32) * STEP + START
            j = j - jnp.where(v < e_lo, 1, 0) + jnp.where(v >= e_lo + STEP, 1, 0)
            j = jnp.minimum(jnp.maximum(j, 0), NBINS - 1)
            for c in range(C):
                w = wv[pl.ds((n * C + c) * rows_per_w + g * LANES, LANES)]
                plsc.addupdate_scatter(accs[c], [rows, j], w)

        def nbody(i, _, step=step):
            step(2 * i)
            step(2 * i + 1)
            return 0

        lax.fori_loop(0, N // 2, nbody, 0)

    for c in range(C):
        pltpu.sync_copy(accs[c], outs[c].at[pl.ds(r0, rows_per_w), :])


def kernel(value, weights):
    mesh = plsc.VectorSubcoreMesh(core_axis_name="c", subcore_axis_name="s")
    nworkers = mesh.num_cores * mesh.num_subcores
    rows_per_w = B // nworkers

    vW = value.reshape(nworkers, rows_per_w, N).swapaxes(1, 2).reshape(-1)
    wW = weights.reshape(nworkers, rows_per_w, N * C).swapaxes(1, 2).reshape(-1)

    run = pl.kernel(
        functools.partial(_kde_body, rows_per_w),
        out_type=[jax.ShapeDtypeStruct((B, NBINS), jnp.float32)] * C,
        mesh=mesh,
        compiler_params=pltpu.CompilerParams(needs_layout_passes=False),
        scratch_types=[
            pltpu.VMEM((N * rows_per_w,), jnp.float32),
            pltpu.VMEM((C * N * rows_per_w,), jnp.float32),
        ] + [pltpu.VMEM((rows_per_w, NBINS), jnp.float32)] * C + [
            pltpu.SemaphoreType.DMA,
            pltpu.SemaphoreType.DMA,
        ],
    )
    h0, h1, h2, h3 = run(vW, wW)
    return jnp.stack([h0, h1, h2, h3], axis=2)


# speedup vs baseline: 1.4181x; 1.4181x over previous
import functools

import jax
import jax.numpy as jnp
from jax import lax
from jax.experimental import pallas as pl
from jax.experimental.pallas import tpu as pltpu
from jax.experimental.pallas import tpu_sc as plsc

NBINS = 256
START = -15.0
STEP = 30.0 / 256.0
INV_STEP = 256.0 / 30.0
B, N, C = 1024, 100, 4
LANES = 16
BLK = 128


def _kde_body(rows_per_w, vT_hbm, wT_hbm, o0, o1, o2, o3,
              vv, wv, a0, a1, a2, a3, sem1, sem2):
    nc = 2
    wid = lax.axis_index("s") * nc + lax.axis_index("c")
    r0 = wid * rows_per_w
    c0 = (wid // 4) * BLK
    sub = (wid % 4) * rows_per_w
    outs = (o0, o1, o2, o3)
    accs = (a0, a1, a2, a3)

    cp1 = pltpu.async_copy(vT_hbm.at[:, pl.ds(c0, BLK)], vv, sem1)
    cp2 = pltpu.async_copy(wT_hbm.at[:, pl.ds(c0, BLK)], wv, sem2)

    zeros = jnp.zeros((LANES,), jnp.float32)

    def zbody(r, _):
        for acc in accs:
            for u in range(NBINS // LANES):
                acc[r, pl.ds(u * LANES, LANES)] = zeros
        return 0

    lax.fori_loop(0, rows_per_w, zbody, 0)

    cp1.wait()
    cp2.wait()

    iota = lax.iota(jnp.int32, LANES)
    for g in range(rows_per_w // LANES):
        rows = g * LANES + iota

        def nbody(n, _, g=g, rows=rows):
            v = vv[n, pl.ds(sub + g * LANES, LANES)]
            t = (v - START) * INV_STEP
            j = t.astype(jnp.int32)
            e_lo = j.astype(jnp.float32) * STEP + START
            j = j - jnp.where(v < e_lo, 1, 0) + jnp.where(v >= e_lo + STEP, 1, 0)
            j = jnp.minimum(jnp.maximum(j, 0), NBINS - 1)
            for c in range(C):
                w = wv[n * C + c, pl.ds(sub + g * LANES, LANES)]
                plsc.addupdate_scatter(accs[c], [rows, j], w)
            return 0

        lax.fori_loop(0, N, nbody, 0)

    for c in range(C):
        pltpu.sync_copy(accs[c], outs[c].at[pl.ds(r0, rows_per_w), :])


def kernel(value, weights):
    mesh = plsc.VectorSubcoreMesh(core_axis_name="c", subcore_axis_name="s")
    nworkers = mesh.num_cores * mesh.num_subcores
    rows_per_w = B // nworkers

    vT = value.T
    wT = weights.reshape(B, N * C).T

    run = pl.kernel(
        functools.partial(_kde_body, rows_per_w),
        out_type=[jax.ShapeDtypeStruct((B, NBINS), jnp.float32)] * C,
        mesh=mesh,
        compiler_params=pltpu.CompilerParams(needs_layout_passes=False),
        scratch_types=[
            pltpu.VMEM((N, BLK), jnp.float32),
            pltpu.VMEM((N * C, BLK), jnp.float32),
        ] + [pltpu.VMEM((rows_per_w, NBINS), jnp.float32)] * C + [
            pltpu.SemaphoreType.DMA,
            pltpu.SemaphoreType.DMA,
        ],
    )
    h0, h1, h2, h3 = run(vT, wT)
    return jnp.stack([h0, h1, h2, h3], axis=2)
